# baseline (speedup 1.0000x reference)
import functools

import numpy as np

import jax
import jax.numpy as jnp
from jax.experimental import pallas as pl
from jax.experimental.pallas import tpu as pltpu

_BF16 = jnp.bfloat16
_F32 = jnp.float32


@functools.lru_cache(maxsize=None)
def _interp_mat_np(out_size, in_size):
    out_size, in_size = int(out_size), int(in_size)
    if out_size == in_size:
        return np.eye(out_size, dtype=np.float32)
    scale = in_size / out_size
    src = np.maximum((np.arange(out_size, dtype=np.float64) + 0.5) * scale - 0.5, 0.0)
    i0 = np.clip(np.floor(src).astype(np.int64), 0, in_size - 1)
    i1 = np.minimum(i0 + 1, in_size - 1)
    frac = (src - i0).astype(np.float32)
    m = np.zeros((out_size, in_size), dtype=np.float32)
    rows = np.arange(out_size)
    np.add.at(m, (rows, i0), 1.0 - frac)
    np.add.at(m, (rows, i1), frac)
    return m


@functools.lru_cache(maxsize=None)
def _kron_resize_np(in_hw, out_hw):
    (hin, win), (ho, wo) = in_hw, out_hw
    a = _interp_mat_np(int(ho), int(hin))
    b = _interp_mat_np(int(wo), int(win))
    return np.ascontiguousarray(np.kron(a, b).T)


def _p5p4_body(c1_ref, c2_ref, w0_ref, b0_ref, w1_ref, b1_ref, r_ref,
               p5_ref, p4_ref):
    c1b = c1_ref[0].astype(_BF16)
    y5 = jnp.dot(w0_ref[...], c1b, preferred_element_type=_F32) + b0_ref[...]
    p5_ref[0] = y5.astype(p5_ref.dtype)
    u = jnp.dot(c2_ref[0].astype(_BF16), r_ref[...], preferred_element_type=_F32)
    y4 = jnp.dot(w1_ref[...], u.astype(_BF16), preferred_element_type=_F32) + b1_ref[...]
    p4_ref[0] = y4.astype(p4_ref.dtype)


def _cascade_body(c5_ref, c4_ref, c3_ref,
                  r12_ref, r23_ref, r34_ref,
                  wd1_ref, bd1_ref, wd2_ref, bd2_ref, wd3_ref, bd3_ref,
                  w1a_ref, w1b_ref, b1_ref, w2a_ref, w2b_ref, b2_ref,
                  p1_ref, p2_ref, p3_ref):
    d1 = jnp.dot(wd1_ref[...], c5_ref[0].astype(_BF16),
                 preferred_element_type=_F32) + bd1_ref[...]
    d2 = jnp.dot(wd2_ref[...], c4_ref[0].astype(_BF16),
                 preferred_element_type=_F32) + bd2_ref[...]
    d3 = jnp.dot(wd3_ref[...], c3_ref[0].astype(_BF16),
                 preferred_element_type=_F32) + bd3_ref[...]
    p1 = jnp.dot(d1.astype(_BF16), r12_ref[...], preferred_element_type=_F32)
    d2_2 = jnp.maximum(
        jnp.dot(w1a_ref[...], p1.astype(_BF16), preferred_element_type=_F32)
        + jnp.dot(w1b_ref[...], d2.astype(_BF16), preferred_element_type=_F32)
        + b1_ref[...], 0.0)
    p2 = jnp.dot(d2_2.astype(_BF16), r23_ref[...], preferred_element_type=_F32)
    d3_2 = jnp.maximum(
        jnp.dot(w2a_ref[...], p2.astype(_BF16), preferred_element_type=_F32)
        + jnp.dot(w2b_ref[...], d3.astype(_BF16), preferred_element_type=_F32)
        + b2_ref[...], 0.0)
    p3 = jnp.dot(d3_2.astype(_BF16), r34_ref[...], preferred_element_type=_F32)
    p1_ref[0] = p1.astype(p1_ref.dtype)
    p2_ref[0] = p2.astype(p2_ref.dtype)
    p3_ref[0] = p3.astype(p3_ref.dtype)


def kernel(c1, c2, c3, c4, c5,
           dsn0_w, dsn0_b, dsn1_w, dsn1_b, dsn2_w, dsn2_b, dsn3_w, dsn3_b,
           dsn4_w, dsn4_b, cat0_wa, cat0_wb, cat0_b, cat1_wa, cat1_wb, cat1_b):
    N, ci1, H1, W1 = c1.shape
    _, ci2, H2, W2 = c2.shape
    _, ci3, H3, W3 = c3.shape
    _, ci4, H4, W4 = c4.shape
    _, ci5, H5, W5 = c5.shape
    C = dsn0_w.shape[0]
    dt = c1.dtype
    hw1, hw2, hw3, hw4, hw5 = H1 * W1, H2 * W2, H3 * W3, H4 * W4, H5 * W5

    tile = hw1 if hw1 <= 2048 else 2048
    n_t = hw1 // tile
    r4 = jnp.asarray(_kron_resize_np((H2, W2), (H1, W1)).astype(np.float32)
                     ).astype(_BF16)
    p5f, p4f = pl.pallas_call(
        _p5p4_body,
        out_shape=(jax.ShapeDtypeStruct((N, C, hw1), dt),
                   jax.ShapeDtypeStruct((N, C, hw1), dt)),
        grid=(N, n_t),
        in_specs=[
            pl.BlockSpec((1, ci1, tile), lambda n, j: (n, 0, j)),
            pl.BlockSpec((1, ci2, hw2), lambda n, j: (n, 0, 0)),
            pl.BlockSpec((C, ci1), lambda n, j: (0, 0)),
            pl.BlockSpec((C, 1), lambda n, j: (0, 0)),
            pl.BlockSpec((C, ci2), lambda n, j: (0, 0)),
            pl.BlockSpec((C, 1), lambda n, j: (0, 0)),
            pl.BlockSpec((hw2, tile), lambda n, j: (0, j)),
        ],
        out_specs=(pl.BlockSpec((1, C, tile), lambda n, j: (n, 0, j)),
                   pl.BlockSpec((1, C, tile), lambda n, j: (n, 0, j))),
        compiler_params=pltpu.CompilerParams(
            dimension_semantics=("parallel", "parallel"),
            vmem_limit_bytes=48 * 1024 * 1024),
    )(c1.reshape(N, ci1, hw1), c2.reshape(N, ci2, hw2),
      dsn0_w.astype(_BF16), dsn0_b, dsn1_w.astype(_BF16), dsn1_b, r4)

    r12 = jnp.asarray(_kron_resize_np((H5, W5), (H4, W4))).astype(_BF16)
    r23 = jnp.asarray(_kron_resize_np((H4, W4), (H3, W3))).astype(_BF16)
    r34 = jnp.asarray(_kron_resize_np((H3, W3), (H2, W2))).astype(_BF16)

    def full(shape):
        return pl.BlockSpec(shape, lambda n: (0,) * len(shape))

    p1f, p2f, p3f = pl.pallas_call(
        _cascade_body,
        out_shape=(jax.ShapeDtypeStruct((N, C, hw4), dt),
                   jax.ShapeDtypeStruct((N, C, hw3), dt),
                   jax.ShapeDtypeStruct((N, C, hw2), dt)),
        grid=(N,),
        in_specs=[
            pl.BlockSpec((1, ci5, hw5), lambda n: (n, 0, 0)),
            pl.BlockSpec((1, ci4, hw4), lambda n: (n, 0, 0)),
            pl.BlockSpec((1, ci3, hw3), lambda n: (n, 0, 0)),
            full((hw5, hw4)), full((hw4, hw3)), full((hw3, hw2)),
            full((C, ci5)), full((C, 1)),
            full((C, ci4)), full((C, 1)),
            full((C, ci3)), full((C, 1)),
            full((C, C)), full((C, C)), full((C, 1)),
            full((C, C)), full((C, C)), full((C, 1)),
        ],
        out_specs=(pl.BlockSpec((1, C, hw4), lambda n: (n, 0, 0)),
                   pl.BlockSpec((1, C, hw3), lambda n: (n, 0, 0)),
                   pl.BlockSpec((1, C, hw2), lambda n: (n, 0, 0))),
        compiler_params=pltpu.CompilerParams(
            dimension_semantics=("parallel",),
            vmem_limit_bytes=32 * 1024 * 1024),
    )(c5.reshape(N, ci5, hw5), c4.reshape(N, ci4, hw4), c3.reshape(N, ci3, hw3),
      r12, r23, r34,
      dsn4_w.astype(_BF16), dsn4_b, dsn3_w.astype(_BF16), dsn3_b,
      dsn2_w.astype(_BF16), dsn2_b,
      cat0_wa.astype(_BF16), cat0_wb.astype(_BF16), cat0_b,
      cat1_wa.astype(_BF16), cat1_wb.astype(_BF16), cat1_b)

    return (p1f.reshape(N, C, H4, W4),
            p2f.reshape(N, C, H3, W3),
            p3f.reshape(N, C, H2, W2),
            p4f.reshape(N, C, H1, W1),
            p5f.reshape(N, C, H1, W1))


# speedup vs baseline: 1.2179x; 1.2179x over previous
import functools

import numpy as np

import jax
import jax.numpy as jnp
from jax.experimental import pallas as pl
from jax.experimental.pallas import tpu as pltpu

_BF16 = jnp.bfloat16
_F32 = jnp.float32


@functools.lru_cache(maxsize=None)
def _interp_mat_np(out_size, in_size):
    out_size, in_size = int(out_size), int(in_size)
    if out_size == in_size:
        return np.eye(out_size, dtype=np.float32)
    scale = in_size / out_size
    src = np.maximum((np.arange(out_size, dtype=np.float64) + 0.5) * scale - 0.5, 0.0)
    i0 = np.clip(np.floor(src).astype(np.int64), 0, in_size - 1)
    i1 = np.minimum(i0 + 1, in_size - 1)
    frac = (src - i0).astype(np.float32)
    m = np.zeros((out_size, in_size), dtype=np.float32)
    rows = np.arange(out_size)
    np.add.at(m, (rows, i0), 1.0 - frac)
    np.add.at(m, (rows, i1), frac)
    return m


@functools.lru_cache(maxsize=None)
def _kron_resize_np(in_hw, out_hw):
    (hin, win), (ho, wo) = in_hw, out_hw
    a = _interp_mat_np(int(ho), int(hin))
    b = _interp_mat_np(int(wo), int(win))
    return np.ascontiguousarray(np.kron(a, b).T)


def _neck_body(c1_ref, c2_ref, c3_ref, c4_ref, c5_ref,
               r4_ref, r12_ref, r23_ref, r34_ref,
               w0_ref, b0_ref, w1_ref, b1_ref,
               wd1_ref, bd1_ref, wd2_ref, bd2_ref, wd3_ref, bd3_ref,
               w1a_ref, w1b_ref, b1c_ref, w2a_ref, w2b_ref, b2c_ref,
               p1_ref, p2_ref, p3_ref, p4_ref, p5_ref):
    y5 = jnp.dot(w0_ref[...], c1_ref[0].astype(_BF16),
                 preferred_element_type=_F32) + b0_ref[...]
    p5_ref[0] = y5.astype(p5_ref.dtype)

    u = jnp.dot(c2_ref[0].astype(_BF16), r4_ref[...],
                preferred_element_type=_F32)
    y4 = jnp.dot(w1_ref[...], u.astype(_BF16),
                 preferred_element_type=_F32) + b1_ref[...]
    p4_ref[0] = y4.astype(p4_ref.dtype)

    d1 = jnp.dot(wd1_ref[...], c5_ref[0].astype(_BF16),
                 preferred_element_type=_F32) + bd1_ref[...]
    d2 = jnp.dot(wd2_ref[...], c4_ref[0].astype(_BF16),
                 preferred_element_type=_F32) + bd2_ref[...]
    d3 = jnp.dot(wd3_ref[...], c3_ref[0].astype(_BF16),
                 preferred_element_type=_F32) + bd3_ref[...]
    p1 = jnp.dot(d1.astype(_BF16), r12_ref[...], preferred_element_type=_F32)
    d2_2 = jnp.maximum(
        jnp.dot(w1a_ref[...], p1.astype(_BF16), preferred_element_type=_F32)
        + jnp.dot(w1b_ref[...], d2.astype(_BF16), preferred_element_type=_F32)
        + b1c_ref[...], 0.0)
    p2 = jnp.dot(d2_2.astype(_BF16), r23_ref[...], preferred_element_type=_F32)
    d3_2 = jnp.maximum(
        jnp.dot(w2a_ref[...], p2.astype(_BF16), preferred_element_type=_F32)
        + jnp.dot(w2b_ref[...], d3.astype(_BF16), preferred_element_type=_F32)
        + b2c_ref[...], 0.0)
    p3 = jnp.dot(d3_2.astype(_BF16), r34_ref[...], preferred_element_type=_F32)
    p1_ref[0] = p1.astype(p1_ref.dtype)
    p2_ref[0] = p2.astype(p2_ref.dtype)
    p3_ref[0] = p3.astype(p3_ref.dtype)


def kernel(c1, c2, c3, c4, c5,
           dsn0_w, dsn0_b, dsn1_w, dsn1_b, dsn2_w, dsn2_b, dsn3_w, dsn3_b,
           dsn4_w, dsn4_b, cat0_wa, cat0_wb, cat0_b, cat1_wa, cat1_wb, cat1_b):
    N, ci1, H1, W1 = c1.shape
    _, ci2, H2, W2 = c2.shape
    _, ci3, H3, W3 = c3.shape
    _, ci4, H4, W4 = c4.shape
    _, ci5, H5, W5 = c5.shape
    C = dsn0_w.shape[0]
    dt = c1.dtype
    hw1, hw2, hw3, hw4, hw5 = H1 * W1, H2 * W2, H3 * W3, H4 * W4, H5 * W5

    r4 = jnp.asarray(_kron_resize_np((H2, W2), (H1, W1))).astype(_BF16)
    r12 = jnp.asarray(_kron_resize_np((H5, W5), (H4, W4))).astype(_BF16)
    r23 = jnp.asarray(_kron_resize_np((H4, W4), (H3, W3))).astype(_BF16)
    r34 = jnp.asarray(_kron_resize_np((H3, W3), (H2, W2))).astype(_BF16)

    def full(shape):
        return pl.BlockSpec(shape, lambda n: (0,) * len(shape))

    p1f, p2f, p3f, p4f, p5f = pl.pallas_call(
        _neck_body,
        out_shape=(jax.ShapeDtypeStruct((N, C, hw4), dt),
                   jax.ShapeDtypeStruct((N, C, hw3), dt),
                   jax.ShapeDtypeStruct((N, C, hw2), dt),
                   jax.ShapeDtypeStruct((N, C, hw1), dt),
                   jax.ShapeDtypeStruct((N, C, hw1), dt)),
        grid=(N,),
        in_specs=[
            pl.BlockSpec((1, ci1, hw1), lambda n: (n, 0, 0)),
            pl.BlockSpec((1, ci2, hw2), lambda n: (n, 0, 0)),
            pl.BlockSpec((1, ci3, hw3), lambda n: (n, 0, 0)),
            pl.BlockSpec((1, ci4, hw4), lambda n: (n, 0, 0)),
            pl.BlockSpec((1, ci5, hw5), lambda n: (n, 0, 0)),
            full((hw2, hw1)), full((hw5, hw4)), full((hw4, hw3)), full((hw3, hw2)),
            full((C, ci1)), full((C, 1)),
            full((C, ci2)), full((C, 1)),
            full((C, ci5)), full((C, 1)),
            full((C, ci4)), full((C, 1)),
            full((C, ci3)), full((C, 1)),
            full((C, C)), full((C, C)), full((C, 1)),
            full((C, C)), full((C, C)), full((C, 1)),
        ],
        out_specs=(pl.BlockSpec((1, C, hw4), lambda n: (n, 0, 0)),
                   pl.BlockSpec((1, C, hw3), lambda n: (n, 0, 0)),
                   pl.BlockSpec((1, C, hw2), lambda n: (n, 0, 0)),
                   pl.BlockSpec((1, C, hw1), lambda n: (n, 0, 0)),
                   pl.BlockSpec((1, C, hw1), lambda n: (n, 0, 0))),
        compiler_params=pltpu.CompilerParams(
            dimension_semantics=("parallel",),
            vmem_limit_bytes=56 * 1024 * 1024),
    )(c1.reshape(N, ci1, hw1), c2.reshape(N, ci2, hw2), c3.reshape(N, ci3, hw3),
      c4.reshape(N, ci4, hw4), c5.reshape(N, ci5, hw5),
      r4, r12, r23, r34,
      dsn0_w.astype(_BF16), dsn0_b, dsn1_w.astype(_BF16), dsn1_b,
      dsn4_w.astype(_BF16), dsn4_b, dsn3_w.astype(_BF16), dsn3_b,
      dsn2_w.astype(_BF16), dsn2_b,
      cat0_wa.astype(_BF16), cat0_wb.astype(_BF16), cat0_b,
      cat1_wa.astype(_BF16), cat1_wb.astype(_BF16), cat1_b)

    return (p1f.reshape(N, C, H4, W4),
            p2f.reshape(N, C, H3, W3),
            p3f.reshape(N, C, H2, W2),
            p4f.reshape(N, C, H1, W1),
            p5f.reshape(N, C, H1, W1))


# speedup vs baseline: 2.9658x; 2.4352x over previous
import functools

import numpy as np

import jax
import jax.numpy as jnp
from jax.experimental import pallas as pl
from jax.experimental.pallas import tpu as pltpu

_BF16 = jnp.bfloat16
_F32 = jnp.float32


@functools.lru_cache(maxsize=None)
def _interp_mat_np(out_size, in_size):
    out_size, in_size = int(out_size), int(in_size)
    if out_size == in_size:
        return np.eye(out_size, dtype=np.float32)
    scale = in_size / out_size
    src = np.maximum((np.arange(out_size, dtype=np.float64) + 0.5) * scale - 0.5, 0.0)
    i0 = np.clip(np.floor(src).astype(np.int64), 0, in_size - 1)
    i1 = np.minimum(i0 + 1, in_size - 1)
    frac = (src - i0).astype(np.float32)
    m = np.zeros((out_size, in_size), dtype=np.float32)
    rows = np.arange(out_size)
    np.add.at(m, (rows, i0), 1.0 - frac)
    np.add.at(m, (rows, i1), frac)
    return m


@functools.lru_cache(maxsize=None)
def _resize_lhs_np(in_hw, out_hw):
    (hin, win), (ho, wo) = in_hw, out_hw
    a = _interp_mat_np(int(ho), int(hin))
    b = _interp_mat_np(int(wo), int(win))
    return np.ascontiguousarray(np.kron(a, b))


def _neck_body(c1_ref, x2_ref, x3_ref, x4_ref, x5_ref,
               r4_ref, r12_ref, r23_ref, r34_ref,
               w0t_ref, b0_ref, w1t_ref, b1_ref,
               wd1t_ref, bd1_ref, wd2t_ref, bd2_ref, wd3t_ref, bd3_ref,
               w1at_ref, w1bt_ref, b1c_ref, w2at_ref, w2bt_ref, b2c_ref,
               p1_ref, p2_ref, p3_ref, p4_ref, p5_ref):
    y5 = jax.lax.dot_general(
        c1_ref[0].astype(_BF16), w0t_ref[...],
        (((0,), (0,)), ((), ())), preferred_element_type=_F32) + b0_ref[...]
    p5_ref[0] = y5.astype(p5_ref.dtype)

    d = jnp.dot(x2_ref[0].astype(_BF16), w1t_ref[...],
                preferred_element_type=_F32) + b1_ref[...]
    y4 = jnp.dot(r4_ref[...], d.astype(_BF16), preferred_element_type=_F32)
    p4_ref[0] = y4.astype(p4_ref.dtype)

    d1 = jnp.dot(x5_ref[0].astype(_BF16), wd1t_ref[...],
                 preferred_element_type=_F32) + bd1_ref[...]
    d2 = jnp.dot(x4_ref[0].astype(_BF16), wd2t_ref[...],
                 preferred_element_type=_F32) + bd2_ref[...]
    d3 = jnp.dot(x3_ref[0].astype(_BF16), wd3t_ref[...],
                 preferred_element_type=_F32) + bd3_ref[...]
    p1 = jnp.dot(r12_ref[...], d1.astype(_BF16), preferred_element_type=_F32)
    d2_2 = jnp.maximum(
        jnp.dot(p1.astype(_BF16), w1at_ref[...], preferred_element_type=_F32)
        + jnp.dot(d2.astype(_BF16), w1bt_ref[...], preferred_element_type=_F32)
        + b1c_ref[...], 0.0)
    p2 = jnp.dot(r23_ref[...], d2_2.astype(_BF16), preferred_element_type=_F32)
    d3_2 = jnp.maximum(
        jnp.dot(p2.astype(_BF16), w2at_ref[...], preferred_element_type=_F32)
        + jnp.dot(d3.astype(_BF16), w2bt_ref[...], preferred_element_type=_F32)
        + b2c_ref[...], 0.0)
    p3 = jnp.dot(r34_ref[...], d3_2.astype(_BF16), preferred_element_type=_F32)
    p1_ref[0] = p1.astype(p1_ref.dtype)
    p2_ref[0] = p2.astype(p2_ref.dtype)
    p3_ref[0] = p3.astype(p3_ref.dtype)


def kernel(c1, c2, c3, c4, c5,
           dsn0_w, dsn0_b, dsn1_w, dsn1_b, dsn2_w, dsn2_b, dsn3_w, dsn3_b,
           dsn4_w, dsn4_b, cat0_wa, cat0_wb, cat0_b, cat1_wa, cat1_wb, cat1_b):
    N, ci1, H1, W1 = c1.shape
    _, ci2, H2, W2 = c2.shape
    _, ci3, H3, W3 = c3.shape
    _, ci4, H4, W4 = c4.shape
    _, ci5, H5, W5 = c5.shape
    C = dsn0_w.shape[0]
    dt = c1.dtype
    hw1, hw2, hw3, hw4, hw5 = H1 * W1, H2 * W2, H3 * W3, H4 * W4, H5 * W5

    x2 = c2.transpose(0, 2, 3, 1).reshape(N, hw2, ci2)
    x3 = c3.transpose(0, 2, 3, 1).reshape(N, hw3, ci3)
    x4 = c4.transpose(0, 2, 3, 1).reshape(N, hw4, ci4)
    x5 = c5.transpose(0, 2, 3, 1).reshape(N, hw5, ci5)

    r4 = jnp.asarray(_resize_lhs_np((H2, W2), (H1, W1))).astype(_BF16)
    r12 = jnp.asarray(_resize_lhs_np((H5, W5), (H4, W4))).astype(_BF16)
    r23 = jnp.asarray(_resize_lhs_np((H4, W4), (H3, W3))).astype(_BF16)
    r34 = jnp.asarray(_resize_lhs_np((H3, W3), (H2, W2))).astype(_BF16)

    def full(shape):
        return pl.BlockSpec(shape, lambda n: (0,) * len(shape))

    p1f, p2f, p3f, p4f, p5f = pl.pallas_call(
        _neck_body,
        out_shape=(jax.ShapeDtypeStruct((N, hw4, C), dt),
                   jax.ShapeDtypeStruct((N, hw3, C), dt),
                   jax.ShapeDtypeStruct((N, hw2, C), dt),
                   jax.ShapeDtypeStruct((N, hw1, C), dt),
                   jax.ShapeDtypeStruct((N, hw1, C), dt)),
        grid=(N,),
        in_specs=[
            pl.BlockSpec((1, ci1, hw1), lambda n: (n, 0, 0)),
            pl.BlockSpec((1, hw2, ci2), lambda n: (n, 0, 0)),
            pl.BlockSpec((1, hw3, ci3), lambda n: (n, 0, 0)),
            pl.BlockSpec((1, hw4, ci4), lambda n: (n, 0, 0)),
            pl.BlockSpec((1, hw5, ci5), lambda n: (n, 0, 0)),
            full((hw1, hw2)), full((hw4, hw5)), full((hw3, hw4)), full((hw2, hw3)),
            full((ci1, C)), full((1, C)),
            full((ci2, C)), full((1, C)),
            full((ci5, C)), full((1, C)),
            full((ci4, C)), full((1, C)),
            full((ci3, C)), full((1, C)),
            full((C, C)), full((C, C)), full((1, C)),
            full((C, C)), full((C, C)), full((1, C)),
        ],
        out_specs=(pl.BlockSpec((1, hw4, C), lambda n: (n, 0, 0)),
                   pl.BlockSpec((1, hw3, C), lambda n: (n, 0, 0)),
                   pl.BlockSpec((1, hw2, C), lambda n: (n, 0, 0)),
                   pl.BlockSpec((1, hw1, C), lambda n: (n, 0, 0)),
                   pl.BlockSpec((1, hw1, C), lambda n: (n, 0, 0))),
        compiler_params=pltpu.CompilerParams(
            dimension_semantics=("parallel",),
            vmem_limit_bytes=56 * 1024 * 1024),
    )(c1.reshape(N, ci1, hw1), x2, x3, x4, x5,
      r4, r12, r23, r34,
      dsn0_w.T.astype(_BF16), dsn0_b.T,
      dsn1_w.T.astype(_BF16), dsn1_b.T,
      dsn4_w.T.astype(_BF16), dsn4_b.T,
      dsn3_w.T.astype(_BF16), dsn3_b.T,
      dsn2_w.T.astype(_BF16), dsn2_b.T,
      cat0_wa.T.astype(_BF16), cat0_wb.T.astype(_BF16), cat0_b.T,
      cat1_wa.T.astype(_BF16), cat1_wb.T.astype(_BF16), cat1_b.T)

    def to_nchw(p, h, w):
        return p.reshape(N, h, w, C).transpose(0, 3, 1, 2)

    return (to_nchw(p1f, H4, W4),
            to_nchw(p2f, H3, W3),
            to_nchw(p3f, H2, W2),
            to_nchw(p4f, H1, W1),
            to_nchw(p5f, H1, W1))
